# baseline (speedup 1.0000x reference)
import functools

import jax
import jax.numpy as jnp
from jax import lax
from jax.experimental import pallas as pl
from jax.experimental.pallas import tpu as pltpu
from jax.experimental.pallas import tpu_sc as plsc

DIM = 32
NWORDS = 16384
NCTX = 10
NCTX_PAD = 16
NC, NS = 2, 16
NW = NC * NS
B_PER_W = NWORDS // NW
CHUNK = 128
NCHUNK = B_PER_W // CHUNK
EPS = 1e-8


def _sc_gather_body(word_tbl, widx, ctx_tbl, cidx, out_words, out_ctx,
                    idx_v, rows_v, cidx_v, crows_v, sem, csem):
    wid = lax.axis_index("s") * NC + lax.axis_index("c")
    pltpu.sync_copy(widx.at[pl.ds(wid * NCHUNK, NCHUNK)], idx_v)

    ctx_copy = pltpu.make_async_copy(ctx_tbl.at[cidx_v], crows_v, csem)

    @pl.when(wid == 0)
    def _():
        pltpu.sync_copy(cidx, cidx_v)
        ctx_copy.start()

    copies = []
    for j in range(NCHUNK):
        copies.append(
            pltpu.async_copy(word_tbl.at[idx_v.at[j]], rows_v.at[j], sem))
    for c in copies:
        c.wait()
    pltpu.sync_copy(rows_v, out_words.at[pl.ds(wid * NCHUNK, NCHUNK)])

    @pl.when(wid == 0)
    def _():
        ctx_copy.wait()
        pltpu.sync_copy(crows_v, out_ctx)


@functools.cache
def _sc_gather():
    return functools.partial(
        pl.kernel,
        out_type=(
            jax.ShapeDtypeStruct((NW * NCHUNK, CHUNK, DIM), jnp.float32),
            jax.ShapeDtypeStruct((NCTX_PAD, DIM), jnp.float32),
        ),
        mesh=plsc.VectorSubcoreMesh(core_axis_name="c", subcore_axis_name="s",
                                    num_cores=NC, num_subcores=NS),
        scratch_types=[
            pltpu.VMEM((NCHUNK, CHUNK), jnp.int32),
            pltpu.VMEM((NCHUNK, CHUNK, DIM), jnp.float32),
            pltpu.VMEM((NCTX_PAD,), jnp.int32),
            pltpu.VMEM((NCTX_PAD, DIM), jnp.float32),
            pltpu.SemaphoreType.DMA,
            pltpu.SemaphoreType.DMA,
        ],
        compiler_params=pltpu.CompilerParams(use_tc_tiling_on_sc=False),
    )(_sc_gather_body)


def _tc_dense_body(ctx_ref, e_ref, w1_ref, w2_ref, out_ref):
    h1 = jnp.zeros((1, DIM), jnp.float32)
    for i in range(NCTX):
        w1_blk = w1_ref[:, i * DIM:(i + 1) * DIM]
        h1 = h1 + lax.dot_general(
            ctx_ref[i:i + 1, :], w1_blk,
            (((1,), (1,)), ((), ())),
            preferred_element_type=jnp.float32)
    h2 = lax.dot_general(
        e_ref[...], w2_ref[...],
        (((1,), (1,)), ((), ())),
        preferred_element_type=jnp.float32)
    num = jnp.sum(h2 * h1, axis=1, keepdims=True)
    n1 = jnp.sqrt(jnp.sum(h1 * h1))
    n2 = jnp.sqrt(jnp.sum(h2 * h2, axis=1, keepdims=True))
    denom = jnp.maximum(n1, EPS) * jnp.maximum(n2, EPS)
    out_ref[...] = num / denom


_tc_dense = pl.pallas_call(
    _tc_dense_body,
    out_shape=jax.ShapeDtypeStruct((NWORDS, 1), jnp.float32),
)


def kernel(context, words, ctx_table, word_table, W1, W2):
    cidx = jnp.zeros((NCTX_PAD,), jnp.int32).at[:NCTX].set(
        context.astype(jnp.int32))
    widx = words.astype(jnp.int32).reshape(NW * NCHUNK, CHUNK)
    e3, ctx_rows = _sc_gather()(word_table, widx, ctx_table, cidx)
    e = e3.reshape(NWORDS, DIM)
    score = _tc_dense(ctx_rows, e, W1, W2)
    return score.reshape(NWORDS)


# speedup vs baseline: 1.7084x; 1.7084x over previous
import functools

import jax
import jax.numpy as jnp
from jax import lax
from jax.experimental import pallas as pl
from jax.experimental.pallas import tpu as pltpu
from jax.experimental.pallas import tpu_sc as plsc

DIM = 32
NWORDS = 16384
NCTX = 10
NC, NS = 2, 16
NW = NC * NS
B_PER_W = NWORDS // NW
CHUNK = 128
NCHUNK = B_PER_W // CHUNK
EPS = 1e-8


def _sc_gather_body(word_tbl, widx, out_words, idx_v, rows_v, sem):
    wid = lax.axis_index("s") * NC + lax.axis_index("c")
    pltpu.sync_copy(widx.at[pl.ds(wid * NCHUNK, NCHUNK)], idx_v)
    copies = []
    for j in range(NCHUNK):
        copies.append(
            pltpu.async_copy(word_tbl.at[idx_v.at[j]], rows_v.at[j], sem))
    for c in copies:
        c.wait()
    pltpu.sync_copy(rows_v, out_words.at[pl.ds(wid * NCHUNK, NCHUNK)])


@functools.cache
def _sc_gather():
    return functools.partial(
        pl.kernel,
        out_type=jax.ShapeDtypeStruct((NW * NCHUNK, CHUNK, DIM), jnp.float32),
        mesh=plsc.VectorSubcoreMesh(core_axis_name="c", subcore_axis_name="s",
                                    num_cores=NC, num_subcores=NS),
        scratch_types=[
            pltpu.VMEM((NCHUNK, CHUNK), jnp.int32),
            pltpu.VMEM((NCHUNK, CHUNK, DIM), jnp.float32),
            pltpu.SemaphoreType.DMA,
        ],
        compiler_params=pltpu.CompilerParams(use_tc_tiling_on_sc=False),
    )(_sc_gather_body)


def _tc_dense_body(cidx_ref, ctx_tbl_ref, e_ref, w1t_ref, w2_ref, out_ref,
                   ccols, sem):
    copies = []
    for i in range(NCTX):
        blk = (cidx_ref[i] // 128) * 128
        copies.append(pltpu.async_copy(
            ctx_tbl_ref.at[:, pl.ds(blk, 128)],
            ccols.at[i], sem))
    for c in copies:
        c.wait()
    h1 = jnp.zeros((1, DIM), jnp.float32)
    lane = jax.lax.broadcasted_iota(jnp.int32, (1, 128), 1)
    for i in range(NCTX):
        off = cidx_ref[i] % 128
        col = jnp.sum(jnp.where(lane == off, ccols[i], 0.0),
                      axis=1, keepdims=True)
        colb = col.astype(jnp.bfloat16).astype(jnp.float32)
        blk = w1t_ref[i * DIM:(i + 1) * DIM, :]
        blkb = blk.astype(jnp.bfloat16).astype(jnp.float32)
        h1 = h1 + jnp.sum(blkb * colb, axis=0, keepdims=True)
    h2 = lax.dot_general(
        e_ref[...].astype(jnp.bfloat16),
        w2_ref[...].astype(jnp.bfloat16),
        (((1,), (1,)), ((), ())),
        preferred_element_type=jnp.float32)
    num = jnp.sum(h2 * h1, axis=1, keepdims=True)
    n1 = jnp.sqrt(jnp.sum(h1 * h1))
    n2 = jnp.sqrt(jnp.sum(h2 * h2, axis=1, keepdims=True))
    denom = jnp.maximum(n1, EPS) * jnp.maximum(n2, EPS)
    out_ref[...] = num / denom


_tc_dense = pl.pallas_call(
    _tc_dense_body,
    in_specs=[
        pl.BlockSpec(memory_space=pltpu.SMEM),
        pl.BlockSpec(memory_space=pl.ANY),
        pl.BlockSpec(memory_space=pltpu.VMEM),
        pl.BlockSpec(memory_space=pltpu.VMEM),
        pl.BlockSpec(memory_space=pltpu.VMEM),
    ],
    out_specs=pl.BlockSpec(memory_space=pltpu.VMEM),
    scratch_shapes=[
        pltpu.VMEM((NCTX, DIM, 128), jnp.float32),
        pltpu.SemaphoreType.DMA,
    ],
    out_shape=jax.ShapeDtypeStruct((NWORDS, 1), jnp.float32),
)


def kernel(context, words, ctx_table, word_table, W1, W2):
    widx = words.astype(jnp.int32).reshape(NW * NCHUNK, CHUNK)
    e3 = _sc_gather()(word_table, widx)
    e = e3.reshape(NWORDS, DIM)
    score = _tc_dense(context.astype(jnp.int32), ctx_table.T, e, W1.T, W2)
    return score.reshape(NWORDS)
